# Initial kernel scaffold; baseline (speedup 1.0000x reference)
import functools

import jax
import jax.numpy as jnp
from jax import lax
from jax.experimental import pallas as pl
from jax.experimental.pallas import tpu as pltpu
from jax.experimental.pallas import tpu_sc as plsc

NC = 2
NS = 16
NW = NC * NS
CH = 80


def _sc_degrees(src, dst, n):
    e = src.shape[0]
    ew = e // NW
    nchunk = ew // CH
    rows_per_tile = n // NS
    ones = jnp.ones((CH, 16), jnp.float32)
    zeros = jnp.zeros((rows_per_tile, 16), jnp.float32)

    mesh = plsc.VectorSubcoreMesh(core_axis_name="c", subcore_axis_name="s")

    @functools.partial(
        pl.kernel,
        mesh=mesh,
        out_type=jax.ShapeDtypeStruct((NC, 2, n, 16), jnp.float32),
        scratch_types=[
            pltpu.VMEM((CH,), jnp.int32),
            pltpu.VMEM((CH, 16), jnp.float32),
            pltpu.VMEM((rows_per_tile, 16), jnp.float32),
            pltpu.VMEM_SHARED((n, 16), jnp.float32),
            pltpu.VMEM_SHARED((n, 16), jnp.float32),
        ],
    )
    def k(src_hbm, dst_hbm, ones_hbm, zeros_hbm, out_hbm,
          idx_v, ones_v, stage_v, acc_src, acc_dst):
        c = lax.axis_index("c")
        s = lax.axis_index("s")
        w = s * NC + c
        base = w * ew
        row0 = s * rows_per_tile

        pltpu.sync_copy(ones_hbm, ones_v)
        pltpu.sync_copy(zeros_hbm, stage_v)
        pltpu.sync_copy(stage_v, acc_src.at[pl.ds(row0, rows_per_tile)])
        pltpu.sync_copy(stage_v, acc_dst.at[pl.ds(row0, rows_per_tile)])
        plsc.subcore_barrier()

        def chunk(i, carry):
            off = base + i * CH
            pltpu.sync_copy(src_hbm.at[pl.ds(off, CH)], idx_v)
            pltpu.sync_copy(ones_v, acc_src.at[idx_v], add=True)
            pltpu.sync_copy(dst_hbm.at[pl.ds(off, CH)], idx_v)
            pltpu.sync_copy(ones_v, acc_dst.at[idx_v], add=True)
            return carry

        lax.fori_loop(0, nchunk, chunk, 0)
        plsc.subcore_barrier()

        pltpu.sync_copy(acc_src.at[pl.ds(row0, rows_per_tile)], stage_v)
        pltpu.sync_copy(stage_v, out_hbm.at[c, 0, pl.ds(row0, rows_per_tile)])
        pltpu.sync_copy(acc_dst.at[pl.ds(row0, rows_per_tile)], stage_v)
        pltpu.sync_copy(stage_v, out_hbm.at[c, 1, pl.ds(row0, rows_per_tile)])

    return k(src, dst, ones, zeros)


def _sc_edge_agg(h, src, dst):
    n, d = h.shape
    e = src.shape[0]
    ew = e // NW
    nchunk = ew // CH
    rows_per_tile = n // NS
    zrows = rows_per_tile // 5
    zeros = jnp.zeros((zrows, d), jnp.float32)

    mesh = plsc.VectorSubcoreMesh(core_axis_name="c", subcore_axis_name="s")

    @functools.partial(
        pl.kernel,
        mesh=mesh,
        out_type=jax.ShapeDtypeStruct((NC, n, d), jnp.float32),
        scratch_types=[
            pltpu.VMEM((CH,), jnp.int32),
            pltpu.VMEM((CH,), jnp.int32),
            pltpu.VMEM((CH, d), jnp.float32),
            pltpu.VMEM((zrows, d), jnp.float32),
            pltpu.VMEM_SHARED((n, d), jnp.float32),
            pltpu.SemaphoreType.DMA,
        ],
    )
    def k(h_hbm, src_hbm, dst_hbm, zeros_hbm, out_hbm,
          sidx_v, didx_v, rows_v, stage_v, acc, sem):
        c = lax.axis_index("c")
        s = lax.axis_index("s")
        w = s * NC + c
        base = w * ew
        row0 = s * rows_per_tile

        pltpu.sync_copy(zeros_hbm, stage_v)
        for j in range(5):
            pltpu.sync_copy(stage_v, acc.at[pl.ds(row0 + j * zrows, zrows)])
        plsc.subcore_barrier()

        def chunk(i, carry):
            off = base + i * CH
            pltpu.sync_copy(src_hbm.at[pl.ds(off, CH)], sidx_v)
            pltpu.async_copy(h_hbm.at[sidx_v], rows_v, sem).wait()
            pltpu.sync_copy(dst_hbm.at[pl.ds(off, CH)], didx_v)
            pltpu.sync_copy(rows_v, acc.at[didx_v], add=True)
            return carry

        lax.fori_loop(0, nchunk, chunk, 0)
        plsc.subcore_barrier()

        for j in range(5):
            r = row0 + j * zrows
            pltpu.sync_copy(acc.at[pl.ds(r, zrows)], stage_v)
            pltpu.sync_copy(stage_v, out_hbm.at[c, pl.ds(r, zrows)])

    return k(h, src, dst, zeros)


_DOT = functools.partial(
    lax.dot_general,
    dimension_numbers=(((1,), (0,)), ((), ())),
    preferred_element_type=jnp.float32,
    precision=lax.Precision.HIGHEST,
)


def _norm_col(d0_ref, d1_ref):
    deg = d0_ref[:, 0:1] + d1_ref[:, 0:1]
    return lax.rsqrt(jnp.maximum(deg, 1.0))


def _tc_pre(x, do0, do1, W1):
    n, d = x.shape
    blk = 2000
    grid = n // blk

    def body(x_ref, d0_ref, d1_ref, w_ref, o_ref):
        h = x_ref[...] * _norm_col(d0_ref, d1_ref)
        o_ref[...] = _DOT(h, w_ref[...])

    return pl.pallas_call(
        body,
        grid=(grid,),
        in_specs=[
            pl.BlockSpec((blk, d), lambda i: (i, 0)),
            pl.BlockSpec((blk, 16), lambda i: (i, 0)),
            pl.BlockSpec((blk, 16), lambda i: (i, 0)),
            pl.BlockSpec((d, d), lambda i: (0, 0)),
        ],
        out_specs=pl.BlockSpec((blk, d), lambda i: (i, 0)),
        out_shape=jax.ShapeDtypeStruct((n, d), jnp.float32),
    )(x, do0, do1, W1)


def _tc_mid(a0, a1, di0, di1, do0, do1, b1, W2):
    n, d = a0.shape
    blk = 2000
    grid = n // blk

    def body(a0_ref, a1_ref, di0_ref, di1_ref, do0_ref, do1_ref,
             b_ref, w_ref, o_ref):
        agg = a0_ref[...] + a1_ref[...]
        h1 = agg * _norm_col(di0_ref, di1_ref) + b_ref[...]
        h1 = h1 * _norm_col(do0_ref, do1_ref)
        o_ref[...] = _DOT(h1, w_ref[...])

    vec = pl.BlockSpec((blk, 16), lambda i: (i, 0))
    mat = pl.BlockSpec((blk, d), lambda i: (i, 0))
    return pl.pallas_call(
        body,
        grid=(grid,),
        in_specs=[mat, mat, vec, vec, vec, vec,
                  pl.BlockSpec((1, d), lambda i: (0, 0)),
                  pl.BlockSpec((d, d), lambda i: (0, 0))],
        out_specs=mat,
        out_shape=jax.ShapeDtypeStruct((n, d), jnp.float32),
    )(a0, a1, di0, di1, do0, do1, b1, W2)


def _tc_final(a0, a1, di0, di1, b2, W3, b3, W4, b4):
    n, d = a0.shape
    dh = W3.shape[1]
    blk = 2000
    grid = n // blk

    def body(a0_ref, a1_ref, di0_ref, di1_ref, b2_ref, w3_ref, b3_ref,
             w4_ref, b4_ref, o_ref):
        agg = a0_ref[...] + a1_ref[...]
        h2 = agg * _norm_col(di0_ref, di1_ref) + b2_ref[...]
        t = jnp.maximum(_DOT(h2, w3_ref[...]) + b3_ref[...], 0.0)
        z = _DOT(t, w4_ref[...]) + b4_ref[...]
        rn = jnp.sqrt(jnp.sum(z * z, axis=1, keepdims=True))
        o_ref[...] = jnp.maximum(jnp.abs(z / rn), 1e-6)

    vec = pl.BlockSpec((blk, 16), lambda i: (i, 0))
    mat = pl.BlockSpec((blk, d), lambda i: (i, 0))
    return pl.pallas_call(
        body,
        grid=(grid,),
        in_specs=[mat, mat, vec, vec,
                  pl.BlockSpec((1, d), lambda i: (0, 0)),
                  pl.BlockSpec((d, dh), lambda i: (0, 0)),
                  pl.BlockSpec((1, dh), lambda i: (0, 0)),
                  pl.BlockSpec((dh, d), lambda i: (0, 0)),
                  pl.BlockSpec((1, d), lambda i: (0, 0))],
        out_specs=mat,
        out_shape=jax.ShapeDtypeStruct((n, d), jnp.float32),
    )(a0, a1, di0, di1, b2, W3, b3, W4, b4)


def kernel(x, edge_index, W1, b1, W2, b2, W3, b3, W4, b4):
    src = edge_index[0]
    dst = edge_index[1]
    n = x.shape[0]

    hist = _sc_degrees(src, dst, n)
    do0, do1 = hist[0, 0], hist[1, 0]
    di0, di1 = hist[0, 1], hist[1, 1]

    h0 = _tc_pre(x, do0, do1, W1)
    agg1 = _sc_edge_agg(h0, src, dst)
    h1 = _tc_mid(agg1[0], agg1[1], di0, di1, do0, do1,
                 b1.reshape(1, -1), W2)
    agg2 = _sc_edge_agg(h1, src, dst)
    return _tc_final(agg2[0], agg2[1], di0, di1,
                     b2.reshape(1, -1), W3, b3.reshape(1, -1), W4,
                     b4.reshape(1, -1))


# speedup vs baseline: 3.7180x; 3.7180x over previous
import functools

import jax
import jax.numpy as jnp
from jax import lax
from jax.experimental import pallas as pl
from jax.experimental.pallas import tpu as pltpu
from jax.experimental.pallas import tpu_sc as plsc

NC = 2
NS = 16
NW = NC * NS
CH = 80


def _sc_count(idx, n, d=128):
    e = idx.shape[0]
    ew = e // NW
    nchunk = ew // CH
    npad = ((n + NS * 40 - 1) // (NS * 40)) * (NS * 40)
    rows_per_tile = npad // NS
    zrows = rows_per_tile // 5
    ones = jnp.ones((CH, d), jnp.float32)
    zeros = jnp.zeros((zrows, d), jnp.float32)

    mesh = plsc.VectorSubcoreMesh(core_axis_name="c", subcore_axis_name="s")

    @functools.partial(
        pl.kernel,
        mesh=mesh,
        out_type=jax.ShapeDtypeStruct((NC, npad, d), jnp.float32),
        scratch_types=[
            pltpu.VMEM((CH,), jnp.int32),
            pltpu.VMEM((CH, d), jnp.float32),
            pltpu.VMEM((zrows, d), jnp.float32),
            pltpu.VMEM_SHARED((npad, d), jnp.float32),
        ],
    )
    def k(idx_hbm, ones_hbm, zeros_hbm, out_hbm, idx_v, ones_v, stage_v, acc):
        c = lax.axis_index("c")
        s = lax.axis_index("s")
        w = s * NC + c
        base = w * ew
        row0 = s * rows_per_tile

        pltpu.sync_copy(ones_hbm, ones_v)
        pltpu.sync_copy(zeros_hbm, stage_v)
        for j in range(5):
            pltpu.sync_copy(stage_v, acc.at[pl.ds(row0 + j * zrows, zrows)])
        plsc.subcore_barrier()

        def chunk(i, carry):
            off = base + i * CH
            pltpu.sync_copy(idx_hbm.at[pl.ds(off, CH)], idx_v)
            pltpu.sync_copy(ones_v, acc.at[idx_v], add=True)
            return carry

        lax.fori_loop(0, nchunk, chunk, 0)
        plsc.subcore_barrier()

        for j in range(5):
            r = row0 + j * zrows
            pltpu.sync_copy(acc.at[pl.ds(r, zrows)], stage_v)
            pltpu.sync_copy(stage_v, out_hbm.at[c, pl.ds(r, zrows)])

    return k(idx, ones, zeros)


def _sc_edge_agg(h, src, dst):
    n, d = h.shape
    e = src.shape[0]
    ew = e // NW
    nchunk = ew // CH
    npad = ((n + NS * 40 - 1) // (NS * 40)) * (NS * 40)
    rows_per_tile = npad // NS
    zrows = rows_per_tile // 5
    zeros = jnp.zeros((zrows, d), jnp.float32)

    mesh = plsc.VectorSubcoreMesh(core_axis_name="c", subcore_axis_name="s")

    @functools.partial(
        pl.kernel,
        mesh=mesh,
        out_type=jax.ShapeDtypeStruct((NC, npad, d), jnp.float32),
        scratch_types=[
            pltpu.VMEM((CH,), jnp.int32),
            pltpu.VMEM((CH,), jnp.int32),
            pltpu.VMEM((CH, d), jnp.float32),
            pltpu.VMEM((zrows, d), jnp.float32),
            pltpu.VMEM_SHARED((npad, d), jnp.float32),
            pltpu.SemaphoreType.DMA,
        ],
    )
    def k(h_hbm, src_hbm, dst_hbm, zeros_hbm, out_hbm,
          sidx_v, didx_v, rows_v, stage_v, acc, sem):
        c = lax.axis_index("c")
        s = lax.axis_index("s")
        w = s * NC + c
        base = w * ew
        row0 = s * rows_per_tile

        pltpu.sync_copy(zeros_hbm, stage_v)
        for j in range(5):
            pltpu.sync_copy(stage_v, acc.at[pl.ds(row0 + j * zrows, zrows)])
        plsc.subcore_barrier()

        def chunk(i, carry):
            off = base + i * CH
            pltpu.sync_copy(src_hbm.at[pl.ds(off, CH)], sidx_v)
            pltpu.async_copy(h_hbm.at[sidx_v], rows_v, sem).wait()
            pltpu.sync_copy(dst_hbm.at[pl.ds(off, CH)], didx_v)
            pltpu.sync_copy(rows_v, acc.at[didx_v], add=True)
            return carry

        lax.fori_loop(0, nchunk, chunk, 0)
        plsc.subcore_barrier()

        for j in range(5):
            r = row0 + j * zrows
            pltpu.sync_copy(acc.at[pl.ds(r, zrows)], stage_v)
            pltpu.sync_copy(stage_v, out_hbm.at[c, pl.ds(r, zrows)])

    return k(h, src, dst, zeros)


_DOT = functools.partial(
    lax.dot_general,
    dimension_numbers=(((1,), (0,)), ((), ())),
    preferred_element_type=jnp.float32,
    precision=lax.Precision.HIGHEST,
)


def _tc_norms(cs0, cs1, cd0, cd1):
    n, d = cs0.shape
    blk = 2000
    grid = n // blk

    def body(cs0_ref, cs1_ref, cd0_ref, cd1_ref, ns_ref, nd_ref):
        ns_ref[...] = lax.rsqrt(jnp.maximum(cs0_ref[...] + cs1_ref[...], 1.0))
        nd_ref[...] = lax.rsqrt(jnp.maximum(cd0_ref[...] + cd1_ref[...], 1.0))

    mat = pl.BlockSpec((blk, d), lambda i: (i, 0))
    return pl.pallas_call(
        body,
        grid=(grid,),
        in_specs=[mat, mat, mat, mat],
        out_specs=[mat, mat],
        out_shape=[jax.ShapeDtypeStruct((n, d), jnp.float32),
                   jax.ShapeDtypeStruct((n, d), jnp.float32)],
    )(cs0, cs1, cd0, cd1)


def _tc_pre(x, nsrc, W1):
    n, d = x.shape
    blk = 2000
    grid = n // blk

    def body(x_ref, ns_ref, w_ref, o_ref):
        o_ref[...] = _DOT(x_ref[...] * ns_ref[...], w_ref[...])

    mat = pl.BlockSpec((blk, d), lambda i: (i, 0))
    return pl.pallas_call(
        body,
        grid=(grid,),
        in_specs=[mat, mat, pl.BlockSpec((d, d), lambda i: (0, 0))],
        out_specs=mat,
        out_shape=jax.ShapeDtypeStruct((n, d), jnp.float32),
    )(x, nsrc, W1)


def _tc_mid(a0, a1, ndst, nsrc, b1, W2):
    n, d = a0.shape
    blk = 2000
    grid = n // blk

    def body(a0_ref, a1_ref, nd_ref, ns_ref, b_ref, w_ref, o_ref):
        h1 = (a0_ref[...] + a1_ref[...]) * nd_ref[...] + b_ref[...]
        o_ref[...] = _DOT(h1 * ns_ref[...], w_ref[...])

    mat = pl.BlockSpec((blk, d), lambda i: (i, 0))
    return pl.pallas_call(
        body,
        grid=(grid,),
        in_specs=[mat, mat, mat, mat,
                  pl.BlockSpec((1, d), lambda i: (0, 0)),
                  pl.BlockSpec((d, d), lambda i: (0, 0))],
        out_specs=mat,
        out_shape=jax.ShapeDtypeStruct((n, d), jnp.float32),
    )(a0, a1, ndst, nsrc, b1, W2)


def _tc_final(a0, a1, ndst, b2, W3, b3, W4, b4):
    n, d = a0.shape
    dh = W3.shape[1]
    blk = 2000
    grid = n // blk

    def body(a0_ref, a1_ref, nd_ref, b2_ref, w3_ref, b3_ref,
             w4_ref, b4_ref, o_ref):
        h2 = (a0_ref[...] + a1_ref[...]) * nd_ref[...] + b2_ref[...]
        t = jnp.maximum(_DOT(h2, w3_ref[...]) + b3_ref[...], 0.0)
        z = _DOT(t, w4_ref[...]) + b4_ref[...]
        rn = jnp.sqrt(jnp.sum(z * z, axis=1, keepdims=True))
        o_ref[...] = jnp.maximum(jnp.abs(z / rn), 1e-6)

    mat = pl.BlockSpec((blk, d), lambda i: (i, 0))
    return pl.pallas_call(
        body,
        grid=(grid,),
        in_specs=[mat, mat, mat,
                  pl.BlockSpec((1, d), lambda i: (0, 0)),
                  pl.BlockSpec((d, dh), lambda i: (0, 0)),
                  pl.BlockSpec((1, dh), lambda i: (0, 0)),
                  pl.BlockSpec((dh, d), lambda i: (0, 0)),
                  pl.BlockSpec((1, d), lambda i: (0, 0))],
        out_specs=mat,
        out_shape=jax.ShapeDtypeStruct((n, d), jnp.float32),
    )(a0, a1, ndst, b2, W3, b3, W4, b4)


def kernel(x, edge_index, W1, b1, W2, b2, W3, b3, W4, b4):
    src = edge_index[0]
    dst = edge_index[1]
    n = x.shape[0]

    csrc = _sc_count(src, n)
    cdst = _sc_count(dst, n)
    nsrc, ndst = _tc_norms(csrc[0, :n], csrc[1, :n],
                           cdst[0, :n], cdst[1, :n])

    h0 = _tc_pre(x, nsrc, W1)
    agg1 = _sc_edge_agg(h0, src, dst)
    h1 = _tc_mid(agg1[0, :n], agg1[1, :n], ndst, nsrc,
                 b1.reshape(1, -1), W2)
    agg2 = _sc_edge_agg(h1, src, dst)
    return _tc_final(agg2[0, :n], agg2[1, :n], ndst,
                     b2.reshape(1, -1), W3, b3.reshape(1, -1), W4,
                     b4.reshape(1, -1))


# speedup vs baseline: 6.0175x; 1.6185x over previous
import functools

import jax
import jax.numpy as jnp
from jax import lax
from jax.experimental import pallas as pl
from jax.experimental.pallas import tpu as pltpu
from jax.experimental.pallas import tpu_sc as plsc

NC = 2
NS = 16
NW = NC * NS
CH = 64


NB = 5


def _sc_count(idx1d, n, d=128):
    e = idx1d.shape[0]
    ew = e // NW
    nchunk = ew // CH
    nround = nchunk // NB
    npad = ((n + NS * 40 - 1) // (NS * 40)) * (NS * 40)
    rows_per_tile = npad // NS
    zrows = CH
    nz = rows_per_tile // zrows
    ones = jnp.ones((CH, d), jnp.float32)
    zeros = jnp.zeros((zrows, d), jnp.float32)

    mesh = plsc.VectorSubcoreMesh(core_axis_name="c", subcore_axis_name="s")

    @functools.partial(
        pl.kernel,
        mesh=mesh,
        out_type=jax.ShapeDtypeStruct((NC, npad, d), jnp.float32),
        scratch_types=[
            pltpu.VMEM((CH, d), jnp.float32),
            pltpu.VMEM((zrows, d), jnp.float32),
            pltpu.VMEM_SHARED((npad, d), jnp.float32),
            pltpu.SemaphoreType.DMA,
            pltpu.SemaphoreType.DMA,
        ]
        + [pltpu.VMEM((CH,), jnp.int32) for _ in range(NB)],
    )
    def k(idx_hbm, ones_hbm, zeros_hbm, out_hbm, ones_v, stage_v,
          acc, sem_i, sem_s, *idxb):
        c = lax.axis_index("c")
        s = lax.axis_index("s")
        w = s * NC + c
        base = w * ew
        row0 = s * rows_per_tile

        pltpu.sync_copy(ones_hbm, ones_v)
        pltpu.sync_copy(zeros_hbm, stage_v)
        for j in range(nz):
            pltpu.sync_copy(stage_v, acc.at[pl.ds(row0 + j * zrows, zrows)])
        plsc.subcore_barrier()

        def round_(g, carry):
            ch0 = g * NB
            ih = []
            for b in range(NB):
                off = base + (ch0 + b) * CH
                ih.append(pltpu.async_copy(
                    idx_hbm.at[pl.ds(off, CH)], idxb[b], sem_i))
            for h_ in ih:
                h_.wait()
            sh = []
            for b in range(NB):
                sh.append(pltpu.async_copy(
                    ones_v, acc.at[idxb[b]], sem_s, add=True))
            for b in range(NB):
                sh[b].wait()
            return carry

        lax.fori_loop(0, nround, round_, 0)
        plsc.subcore_barrier()

        for j in range(nz):
            r = row0 + j * zrows
            pltpu.sync_copy(acc.at[pl.ds(r, zrows)], stage_v)
            pltpu.sync_copy(stage_v, out_hbm.at[c, pl.ds(r, zrows)])

    return k(idx1d, ones, zeros)


def _sc_edge_agg(h, src1d, dst1d):
    n, d = h.shape
    e = src1d.shape[0]
    ew = e // NW
    nchunk = ew // CH
    nround = nchunk // NB
    npad = ((n + NS * 40 - 1) // (NS * 40)) * (NS * 40)
    rows_per_tile = npad // NS
    zrows = CH
    nz = rows_per_tile // zrows
    zeros = jnp.zeros((zrows, d), jnp.float32)

    mesh = plsc.VectorSubcoreMesh(core_axis_name="c", subcore_axis_name="s")

    @functools.partial(
        pl.kernel,
        mesh=mesh,
        out_type=jax.ShapeDtypeStruct((NC, npad, d), jnp.float32),
        scratch_types=[
            pltpu.VMEM_SHARED((npad, d), jnp.float32),
            pltpu.SemaphoreType.DMA,
            pltpu.SemaphoreType.DMA,
        ]
        + [pltpu.SemaphoreType.DMA for _ in range(NB)]
        + [pltpu.VMEM((CH,), jnp.int32) for _ in range(2 * NB)]
        + [pltpu.VMEM((CH, d), jnp.float32) for _ in range(NB)],
    )
    def k(h_hbm, src_hbm, dst_hbm, zeros_hbm, out_hbm,
          acc, sem_i, sem_s, *bufs):
        sem_g = bufs[:NB]
        sidx = bufs[NB:2 * NB]
        didx = bufs[2 * NB:3 * NB]
        rows = bufs[3 * NB:]
        c = lax.axis_index("c")
        s = lax.axis_index("s")
        w = s * NC + c
        base = w * ew
        row0 = s * rows_per_tile

        pltpu.sync_copy(zeros_hbm, rows[0])
        for j in range(nz):
            pltpu.sync_copy(rows[0], acc.at[pl.ds(row0 + j * zrows, zrows)])
        plsc.subcore_barrier()

        def round_(g, carry):
            ch0 = g * NB
            ih = []
            for b in range(NB):
                off = base + (ch0 + b) * CH
                ih.append(pltpu.async_copy(
                    src_hbm.at[pl.ds(off, CH)], sidx[b], sem_i))
                ih.append(pltpu.async_copy(
                    dst_hbm.at[pl.ds(off, CH)], didx[b], sem_i))
            for h_ in ih:
                h_.wait()
            gh = []
            for b in range(NB):
                gh.append(pltpu.async_copy(
                    h_hbm.at[sidx[b]], rows[b], sem_g[b]))
            sh = []
            for b in range(NB):
                gh[b].wait()
                sh.append(pltpu.async_copy(
                    rows[b], acc.at[didx[b]], sem_s, add=True))
            for b in range(NB):
                sh[b].wait()
            return carry

        lax.fori_loop(0, nround, round_, 0)
        plsc.subcore_barrier()

        for j in range(nz):
            r = row0 + j * zrows
            pltpu.sync_copy(acc.at[pl.ds(r, zrows)], rows[0])
            pltpu.sync_copy(rows[0], out_hbm.at[c, pl.ds(r, zrows)])

    return k(h, src1d, dst1d, zeros)


_DOT = functools.partial(
    lax.dot_general,
    dimension_numbers=(((1,), (0,)), ((), ())),
    preferred_element_type=jnp.float32,
    precision=lax.Precision.HIGHEST,
)


def _tc_norms(cs0, cs1, cd0, cd1):
    n, d = cs0.shape
    blk = 2000
    grid = n // blk

    def body(cs0_ref, cs1_ref, cd0_ref, cd1_ref, ns_ref, nd_ref):
        ns_ref[...] = lax.rsqrt(jnp.maximum(cs0_ref[...] + cs1_ref[...], 1.0))
        nd_ref[...] = lax.rsqrt(jnp.maximum(cd0_ref[...] + cd1_ref[...], 1.0))

    mat = pl.BlockSpec((blk, d), lambda i: (i, 0))
    return pl.pallas_call(
        body,
        grid=(grid,),
        in_specs=[mat, mat, mat, mat],
        out_specs=[mat, mat],
        out_shape=[jax.ShapeDtypeStruct((n, d), jnp.float32),
                   jax.ShapeDtypeStruct((n, d), jnp.float32)],
    )(cs0, cs1, cd0, cd1)


def _tc_pre(x, nsrc, W1):
    n, d = x.shape
    blk = 2000
    grid = n // blk

    def body(x_ref, ns_ref, w_ref, o_ref):
        o_ref[...] = _DOT(x_ref[...] * ns_ref[...], w_ref[...])

    mat = pl.BlockSpec((blk, d), lambda i: (i, 0))
    return pl.pallas_call(
        body,
        grid=(grid,),
        in_specs=[mat, mat, pl.BlockSpec((d, d), lambda i: (0, 0))],
        out_specs=mat,
        out_shape=jax.ShapeDtypeStruct((n, d), jnp.float32),
    )(x, nsrc, W1)


def _tc_mid(a0, a1, ndst, nsrc, b1, W2):
    n, d = a0.shape
    blk = 2000
    grid = n // blk

    def body(a0_ref, a1_ref, nd_ref, ns_ref, b_ref, w_ref, o_ref):
        h1 = (a0_ref[...] + a1_ref[...]) * nd_ref[...] + b_ref[...]
        o_ref[...] = _DOT(h1 * ns_ref[...], w_ref[...])

    mat = pl.BlockSpec((blk, d), lambda i: (i, 0))
    return pl.pallas_call(
        body,
        grid=(grid,),
        in_specs=[mat, mat, mat, mat,
                  pl.BlockSpec((1, d), lambda i: (0, 0)),
                  pl.BlockSpec((d, d), lambda i: (0, 0))],
        out_specs=mat,
        out_shape=jax.ShapeDtypeStruct((n, d), jnp.float32),
    )(a0, a1, ndst, nsrc, b1, W2)


def _tc_final(a0, a1, ndst, b2, W3, b3, W4, b4):
    n, d = a0.shape
    dh = W3.shape[1]
    blk = 2000
    grid = n // blk

    def body(a0_ref, a1_ref, nd_ref, b2_ref, w3_ref, b3_ref,
             w4_ref, b4_ref, o_ref):
        h2 = (a0_ref[...] + a1_ref[...]) * nd_ref[...] + b2_ref[...]
        t = jnp.maximum(_DOT(h2, w3_ref[...]) + b3_ref[...], 0.0)
        z = _DOT(t, w4_ref[...]) + b4_ref[...]
        rn = jnp.sqrt(jnp.sum(z * z, axis=1, keepdims=True))
        o_ref[...] = jnp.maximum(jnp.abs(z / rn), 1e-6)

    mat = pl.BlockSpec((blk, d), lambda i: (i, 0))
    return pl.pallas_call(
        body,
        grid=(grid,),
        in_specs=[mat, mat, mat,
                  pl.BlockSpec((1, d), lambda i: (0, 0)),
                  pl.BlockSpec((d, dh), lambda i: (0, 0)),
                  pl.BlockSpec((1, dh), lambda i: (0, 0)),
                  pl.BlockSpec((dh, d), lambda i: (0, 0)),
                  pl.BlockSpec((1, d), lambda i: (0, 0))],
        out_specs=mat,
        out_shape=jax.ShapeDtypeStruct((n, d), jnp.float32),
    )(a0, a1, ndst, b2, W3, b3, W4, b4)


def kernel(x, edge_index, W1, b1, W2, b2, W3, b3, W4, b4):
    n = x.shape[0]
    e = edge_index.shape[1]

    npad = ((n + NS * 40 - 1) // (NS * 40)) * (NS * 40)
    quant = NW * CH * NB
    epad = ((e + quant - 1) // quant) * quant
    pad = epad - e
    src1d = edge_index[0]
    dst1d = edge_index[1]
    src_cnt = src1d
    if pad:
        fill_gather = (jnp.arange(pad, dtype=jnp.int32) * 97) % n
        fill_discard = n + (jnp.arange(pad, dtype=jnp.int32) % (npad - n))
        src1d = jnp.concatenate([src1d, fill_gather])
        src_cnt = jnp.concatenate([src_cnt, fill_discard])
        dst1d = jnp.concatenate([dst1d, fill_discard])

    csrc = _sc_count(src_cnt, n)
    cdst = _sc_count(dst1d, n)
    nsrc, ndst = _tc_norms(csrc[0, :n], csrc[1, :n],
                           cdst[0, :n], cdst[1, :n])

    h0 = _tc_pre(x, nsrc, W1)
    agg1 = _sc_edge_agg(h0, src1d, dst1d)
    h1 = _tc_mid(agg1[0, :n], agg1[1, :n], ndst, nsrc,
                 b1.reshape(1, -1), W2)
    agg2 = _sc_edge_agg(h1, src1d, dst1d)
    return _tc_final(agg2[0, :n], agg2[1, :n], ndst,
                     b2.reshape(1, -1), W3, b3.reshape(1, -1), W4,
                     b4.reshape(1, -1))
